# baseline (speedup 1.0000x reference)
import functools

import jax
import jax.numpy as jnp
from jax import lax
from jax.experimental import pallas as pl
from jax.experimental.pallas import tpu as pltpu
from jax.experimental.pallas import tpu_sc as plsc

N = 10000
E = 320000
D = 128
NP = 10240

NC = 1
NS = 16
NW = NC * NS
EPW = E // NW
LANES = 16
STEPS = EPW // LANES

BN = 5120
GRID = NP // BN
BNC = 2048
GRIDC = NP // BNC
ECH0 = (EPW // 32) * 16


def _mlp_body(x_ref, w0_ref, b0_ref, a0_ref, w1_ref, a1_ref, w2_ref, a2_ref,
              w3_ref, b3_ref, edge_ref, out_row_ref, out_lin_ref,
              row_lin_ref, col_lin_ref, semr, semc, seml):
    i = pl.program_id(0)
    cr = pltpu.make_async_copy(edge_ref.at[0], row_lin_ref, semr)
    cc = pltpu.make_async_copy(edge_ref.at[1], col_lin_ref, semc)

    @pl.when(i == 0)
    def _():
        cr.start()
        cc.start()

    x = x_ref[...]
    dn = (((1,), (1,)), ((), ()))
    h = lax.dot_general(x, w0_ref[...], dn, preferred_element_type=jnp.float32)
    h = h + b0_ref[...]
    h = jnp.where(h >= 0, h, a0_ref[0, 0] * h)
    h = lax.dot_general(h, w1_ref[...], dn, preferred_element_type=jnp.float32)
    h = jnp.where(h >= 0, h, a1_ref[0, 0] * h)
    h = lax.dot_general(h, w2_ref[...], dn, preferred_element_type=jnp.float32)
    h = jnp.where(h >= 0, h, a2_ref[0, 0] * h)
    ang = lax.dot_general(w3_ref[...], h, dn, preferred_element_type=jnp.float32)
    scaled = 0.25 * (ang + b3_ref[0, 0])
    out_row_ref[:, pl.ds(i * BN, BN)] = scaled

    @pl.when(i == GRID - 1)
    def _():
        cp = pltpu.make_async_copy(out_row_ref.at[0], out_lin_ref, seml)
        cp.start()
        cp.wait()
        cr.wait()
        cc.wait()


def _mlp(coeffs, W0, b0, a0, W1, a1, W2, a2, W3, b3, edge_index):
    full = lambda shape: pl.BlockSpec(shape, lambda i: (0, 0))
    return pl.pallas_call(
        _mlp_body,
        grid=(GRID,),
        in_specs=[
            pl.BlockSpec((BN, D), lambda i: (i, 0)),
            full((D, D)), full((1, D)), full((1, 1)),
            full((D, D)), full((1, 1)),
            full((D, D)), full((1, 1)),
            full((1, D)), full((1, 1)),
            pl.BlockSpec((2, E), lambda i: (0, 0)),
        ],
        out_specs=[
            pl.BlockSpec((1, NP), lambda i: (0, 0)),
            pl.BlockSpec(memory_space=pltpu.MemorySpace.HBM),
            pl.BlockSpec(memory_space=pltpu.MemorySpace.HBM),
            pl.BlockSpec(memory_space=pltpu.MemorySpace.HBM),
        ],
        out_shape=[
            jax.ShapeDtypeStruct((1, NP), jnp.float32),
            jax.ShapeDtypeStruct((NP,), jnp.float32),
            jax.ShapeDtypeStruct((E,), jnp.int32),
            jax.ShapeDtypeStruct((E,), jnp.int32),
        ],
        scratch_shapes=[
            pltpu.SemaphoreType.DMA,
            pltpu.SemaphoreType.DMA,
            pltpu.SemaphoreType.DMA,
        ],
    )(coeffs, W0, b0.reshape(1, D), a0.reshape(1, 1),
      W1, a1.reshape(1, 1), W2, a2.reshape(1, 1),
      W3, b3.reshape(1, 1), edge_index)


_NCH = 4
_CH_OFF = tuple((EPW * k // _NCH) // 16 * 16 for k in range(_NCH))
_CH_END = tuple(_CH_OFF[1:]) + (EPW,)


def _sc_body(scaled_hbm, row_hbm, col_hbm, out_hbm, tab_v, row_v, col_v, acc_v,
             semt, *sems):
    wid = lax.axis_index("s") * NC + lax.axis_index("c")
    base = wid * EPW
    ct = pltpu.make_async_copy(scaled_hbm, tab_v, semt)
    ct.start()
    copies = []
    for k in range(_NCH):
        o, ln = _CH_OFF[k], _CH_END[k] - _CH_OFF[k]
        cr = pltpu.make_async_copy(row_hbm.at[pl.ds(base + o, ln)],
                                   row_v.at[pl.ds(o, ln)], sems[2 * k])
        cc = pltpu.make_async_copy(col_hbm.at[pl.ds(base + o, ln)],
                                   col_v.at[pl.ds(o, ln)], sems[2 * k + 1])
        cr.start()
        cc.start()
        copies.append((cr, cc))

    zeros = jnp.zeros((LANES,), jnp.float32)

    @plsc.parallel_loop(0, NP // LANES, unroll=16)
    def zbody(i):
        acc_v[pl.ds(i * LANES, LANES)] = zeros

    ct.wait()
    for k in range(_NCH):
        cr, cc = copies[k]
        cr.wait()
        cc.wait()

        @plsc.parallel_loop(_CH_OFF[k] // LANES, _CH_END[k] // LANES,
                            unroll=16)
        def body(i):
            c = col_v[pl.ds(i * LANES, LANES)]
            r = row_v[pl.ds(i * LANES, LANES)]
            vals = plsc.load_gather(tab_v, [c])
            plsc.addupdate_scatter(acc_v, [r], vals)

    pltpu.sync_copy(acc_v, out_hbm.at[wid])


def _sc_scatter(scaled_flat, row, col):
    mesh = plsc.VectorSubcoreMesh(core_axis_name="c", subcore_axis_name="s",
                                  num_cores=NC)
    k = functools.partial(
        pl.kernel,
        mesh=mesh,
        out_type=jax.ShapeDtypeStruct((NW, NP), jnp.float32),
        scratch_types=[
            pltpu.VMEM((NP,), jnp.float32),
            pltpu.VMEM((EPW,), jnp.int32),
            pltpu.VMEM((EPW,), jnp.int32),
            pltpu.VMEM((NP,), jnp.float32),
        ] + [pltpu.SemaphoreType.DMA] * (1 + 2 * _NCH),
        compiler_params=pltpu.CompilerParams(needs_layout_passes=False),
    )(_sc_body)
    return k(scaled_flat, row, col)


def _combine_body(p_ref, s_ref, out_ref):
    i = pl.program_id(0)
    sl = pl.ds(i * BNC, BNC)
    v = 4.0 * s_ref[:, sl] + jnp.sum(p_ref[:, sl], axis=0, keepdims=True)

    @pl.when(i < GRIDC - 1)
    def _():
        out_ref[:, pl.ds(i * BNC, BNC)] = v

    @pl.when(i == GRIDC - 1)
    def _():
        out_ref[:, pl.ds(i * BNC, N - (GRIDC - 1) * BNC)] = \
            v[:, :N - (GRIDC - 1) * BNC]


def _combine(partials, scaled_row):
    return pl.pallas_call(
        _combine_body,
        grid=(GRIDC,),
        in_specs=[
            pl.BlockSpec((NW, NP), lambda i: (0, 0)),
            pl.BlockSpec((1, NP), lambda i: (0, 0)),
        ],
        out_specs=pl.BlockSpec((1, N), lambda i: (0, 0)),
        out_shape=jax.ShapeDtypeStruct((1, N), jnp.float32),
    )(partials, scaled_row)


def kernel(coeffs, edge_index, W0, b0, a0, W1, a1, W2, a2, W3, b3):
    scaled_row, scaled_lin, row_lin, col_lin = _mlp(
        coeffs, W0, b0, a0, W1, a1, W2, a2, W3, b3, edge_index)
    partials = _sc_scatter(scaled_lin, row_lin, col_lin)
    return _combine(partials, scaled_row).reshape(N, 1)


# speedup vs baseline: 1.0363x; 1.0363x over previous
import functools

import jax
import jax.numpy as jnp
from jax import lax
from jax.experimental import pallas as pl
from jax.experimental.pallas import tpu as pltpu
from jax.experimental.pallas import tpu_sc as plsc

N = 10000
E = 320000
D = 128
NP = 10240

NC = 1
NS = 16
NW = NC * NS
EPW = E // NW
LANES = 16
STEPS = EPW // LANES

BN = 5120
GRID = NP // BN
BNC = 2048
GRIDC = NP // BNC
ECH0 = (EPW // 32) * 16


def _mlp_body(x_ref, w0_ref, b0_ref, a0_ref, w1_ref, a1_ref, w2_ref, a2_ref,
              w3_ref, b3_ref, edge_ref, out_row_ref,
              row_lin_ref, col_lin_ref, semr, semc):
    i = pl.program_id(0)
    cr = pltpu.make_async_copy(edge_ref.at[0], row_lin_ref, semr)
    cc = pltpu.make_async_copy(edge_ref.at[1], col_lin_ref, semc)

    @pl.when(i == 0)
    def _():
        cr.start()
        cc.start()

    x = x_ref[...]
    dn = (((1,), (1,)), ((), ()))
    h = lax.dot_general(x, w0_ref[...], dn, preferred_element_type=jnp.float32)
    h = h + b0_ref[...]
    h = jnp.where(h >= 0, h, a0_ref[0, 0] * h)
    h = lax.dot_general(h, w1_ref[...], dn, preferred_element_type=jnp.float32)
    h = jnp.where(h >= 0, h, a1_ref[0, 0] * h)
    h = lax.dot_general(h, w2_ref[...], dn, preferred_element_type=jnp.float32)
    h = jnp.where(h >= 0, h, a2_ref[0, 0] * h)
    ang = lax.dot_general(w3_ref[...], h, dn, preferred_element_type=jnp.float32)
    scaled = 0.25 * (ang + b3_ref[0, 0])
    out_row_ref[:, pl.ds(i * BN, BN)] = scaled

    @pl.when(i == GRID - 1)
    def _():
        cr.wait()
        cc.wait()


def _mlp(coeffs, W0, b0, a0, W1, a1, W2, a2, W3, b3, edge_index):
    full = lambda shape: pl.BlockSpec(shape, lambda i: (0, 0))
    return pl.pallas_call(
        _mlp_body,
        grid=(GRID,),
        in_specs=[
            pl.BlockSpec((BN, D), lambda i: (i, 0)),
            full((D, D)), full((1, D)), full((1, 1)),
            full((D, D)), full((1, 1)),
            full((D, D)), full((1, 1)),
            full((1, D)), full((1, 1)),
            pl.BlockSpec((2, E), lambda i: (0, 0)),
        ],
        out_specs=[
            pl.BlockSpec((1, NP), lambda i: (0, 0)),
            pl.BlockSpec(memory_space=pltpu.MemorySpace.HBM),
            pl.BlockSpec(memory_space=pltpu.MemorySpace.HBM),
        ],
        out_shape=[
            jax.ShapeDtypeStruct((1, NP), jnp.float32),
            jax.ShapeDtypeStruct((E,), jnp.int32),
            jax.ShapeDtypeStruct((E,), jnp.int32),
        ],
        scratch_shapes=[
            pltpu.SemaphoreType.DMA,
            pltpu.SemaphoreType.DMA,
        ],
    )(coeffs, W0, b0.reshape(1, D), a0.reshape(1, 1),
      W1, a1.reshape(1, 1), W2, a2.reshape(1, 1),
      W3, b3.reshape(1, 1), edge_index)


_NCH = 2
_CH_OFF = tuple((EPW * k // _NCH) // 16 * 16 for k in range(_NCH))
_CH_END = tuple(_CH_OFF[1:]) + (EPW,)


def _sc_body(scaled_hbm, row_hbm, col_hbm, out_hbm, tab_v, row_v, col_v, acc_v,
             semt, *sems):
    wid = lax.axis_index("s") * NC + lax.axis_index("c")
    base = wid * EPW
    ct = pltpu.make_async_copy(scaled_hbm, tab_v, semt)
    ct.start()
    copies = []
    for k in range(_NCH):
        o, ln = _CH_OFF[k], _CH_END[k] - _CH_OFF[k]
        cr = pltpu.make_async_copy(row_hbm.at[pl.ds(base + o, ln)],
                                   row_v.at[pl.ds(o, ln)], sems[2 * k])
        cc = pltpu.make_async_copy(col_hbm.at[pl.ds(base + o, ln)],
                                   col_v.at[pl.ds(o, ln)], sems[2 * k + 1])
        cr.start()
        cc.start()
        copies.append((cr, cc))

    zeros = jnp.zeros((LANES,), jnp.float32)

    @plsc.parallel_loop(0, NP // LANES, unroll=16)
    def zbody(i):
        acc_v[pl.ds(i * LANES, LANES)] = zeros

    ct.wait()
    for k in range(_NCH):
        cr, cc = copies[k]
        cr.wait()
        cc.wait()

        @plsc.parallel_loop(_CH_OFF[k] // LANES, _CH_END[k] // LANES,
                            unroll=16)
        def body(i):
            c = col_v[pl.ds(i * LANES, LANES)]
            r = row_v[pl.ds(i * LANES, LANES)]
            vals = plsc.load_gather(tab_v, [c])
            plsc.addupdate_scatter(acc_v, [r], vals)

    pltpu.sync_copy(acc_v, out_hbm.at[wid])


def _sc_scatter(scaled_flat, row, col):
    mesh = plsc.VectorSubcoreMesh(core_axis_name="c", subcore_axis_name="s",
                                  num_cores=NC)
    k = functools.partial(
        pl.kernel,
        mesh=mesh,
        out_type=jax.ShapeDtypeStruct((NW, NP), jnp.float32),
        scratch_types=[
            pltpu.VMEM((NP,), jnp.float32),
            pltpu.VMEM((EPW,), jnp.int32),
            pltpu.VMEM((EPW,), jnp.int32),
            pltpu.VMEM((NP,), jnp.float32),
        ] + [pltpu.SemaphoreType.DMA] * (1 + 2 * _NCH),
        compiler_params=pltpu.CompilerParams(needs_layout_passes=False),
    )(_sc_body)
    return k(scaled_flat, row, col)


def _combine_body(p_ref, s_ref, out_ref):
    i = pl.program_id(0)
    sl = pl.ds(i * BNC, BNC)
    v = 4.0 * s_ref[:, sl] + jnp.sum(p_ref[:, sl], axis=0, keepdims=True)

    @pl.when(i < GRIDC - 1)
    def _():
        out_ref[:, pl.ds(i * BNC, BNC)] = v

    @pl.when(i == GRIDC - 1)
    def _():
        out_ref[:, pl.ds(i * BNC, N - (GRIDC - 1) * BNC)] = \
            v[:, :N - (GRIDC - 1) * BNC]


def _combine(partials, scaled_row):
    return pl.pallas_call(
        _combine_body,
        grid=(GRIDC,),
        in_specs=[
            pl.BlockSpec((NW, NP), lambda i: (0, 0)),
            pl.BlockSpec((1, NP), lambda i: (0, 0)),
        ],
        out_specs=pl.BlockSpec((1, N), lambda i: (0, 0)),
        out_shape=jax.ShapeDtypeStruct((1, N), jnp.float32),
    )(partials, scaled_row)


def kernel(coeffs, edge_index, W0, b0, a0, W1, a1, W2, a2, W3, b3):
    scaled_row, row_lin, col_lin = _mlp(
        coeffs, W0, b0, a0, W1, a1, W2, a2, W3, b3, edge_index)
    partials = _sc_scatter(scaled_row.reshape(NP), row_lin, col_lin)
    return _combine(partials, scaled_row).reshape(N, 1)
